# baseline (speedup 1.0000x reference)
import functools

import jax
import jax.numpy as jnp
from jax import lax
from jax.experimental import pallas as pl
from jax.experimental.pallas import tpu as pltpu
from jax.experimental.pallas import tpu_sc as plsc

NC = 2
NS = 16
NW = NC * NS

DIM = 256
CH = 64
NB = 4
LOOK = 2


def _make_lookup(B: int):
  assert B % (NW * CH) == 0
  bpw = B // NW
  iters = bpw // CH
  assert iters % NB == 0 and NB >= LOOK + 2
  mesh = plsc.VectorSubcoreMesh(core_axis_name="c", subcore_axis_name="s")

  @functools.partial(
      pl.kernel,
      mesh=mesh,
      out_type=jax.ShapeDtypeStruct((NW, iters, CH, DIM), jnp.float32),
      scratch_types=[
          pltpu.VMEM((iters, CH), jnp.int32),
          pltpu.VMEM((NB, CH, DIM), jnp.float32),
      ] + [pltpu.SemaphoreType.DMA] * (2 * NB),
  )
  def lookup(codes_hbm, table_hbm, out_hbm, idx_v, rows_v, *sems):
    gsems = sems[:NB]
    osems = sems[NB:]
    wid = lax.axis_index("s") * NC + lax.axis_index("c")

    pltpu.sync_copy(codes_hbm.at[wid], idx_v)

    for b in range(LOOK):
      pltpu.async_copy(table_hbm.at[idx_v.at[b]], rows_v.at[b], gsems[b])

    def group(go, carry):
      for b in range(NB):
        g = go * NB + b

        bl = (b + LOOK) % NB

        @pl.when(jnp.logical_and(g + LOOK - NB >= 0, g + LOOK < iters))
        def _():
          pltpu.make_async_copy(
              rows_v.at[bl], out_hbm.at[wid, g + LOOK - NB],
              osems[bl]).wait()

        @pl.when(g + LOOK < iters)
        def _():
          pltpu.async_copy(
              table_hbm.at[idx_v.at[g + LOOK]], rows_v.at[bl], gsems[bl])

        pltpu.make_async_copy(
            table_hbm.at[idx_v.at[g]], rows_v.at[b], gsems[b]).wait()
        pltpu.async_copy(rows_v.at[b], out_hbm.at[wid, g], osems[b])

      return carry

    lax.fori_loop(0, iters // NB, group, 0)

    for b in range(NB):
      g = iters - NB + b
      pltpu.make_async_copy(
          rows_v.at[g % NB], out_hbm.at[wid, g], osems[g % NB]).wait()

  return lookup


def kernel(codes, table):
  batch, hist = codes.shape
  B = batch * hist
  codes_r = codes.reshape(NW, B // (NW * CH), CH).astype(jnp.int32)
  out = _make_lookup(B)(codes_r, table)
  return out.reshape(batch, hist, DIM)
